# baseline (speedup 1.0000x reference)
import jax
import jax.numpy as jnp
from jax import lax
from jax.experimental import pallas as pl
from jax.experimental.pallas import tpu as pltpu
from jax.experimental.pallas import tpu_sc as plsc

VOCAB = 1000000
BATCH = 16384
FIELDS = 26
EMBED = 64
NUM_WORKERS = 32
NBLK = BATCH // 128
BLK_PER_W = NBLK // NUM_WORKERS
ITEMS = FIELDS * BLK_PER_W
GPAD = 129

NC_FULL = VOCAB // 128
A_ITERS = 124


def _relayout_body(et_hbm, tail_hbm, out_hbm, gbufs, obufs, gsem, ssem):
    wid = lax.axis_index("s") * 2 + lax.axis_index("c")
    start = wid * 244 + lax.min(wid, 4)
    cnt = jnp.where(wid < 4, 245, 244)

    iota = lax.iota(jnp.int32, 16)
    rows4 = [iota + c0 for c0 in (0, 16, 32, 48)]

    def fire_gather(t, b):
        c = pl.multiple_of((start + t) * 128, 128)
        pltpu.async_copy(
            et_hbm.at[:, pl.ds(c, 128)], gbufs[b].at[:, pl.ds(0, 128)], gsem)

    def wait_gather(b):
        pltpu.make_async_copy(
            et_hbm.at[:, pl.ds(0, 128)], gbufs[b].at[:, pl.ds(0, 128)],
            gsem).wait()

    def fire_store(t, b):
        o = pl.multiple_of((start + t) * 8192, 8192)
        pltpu.async_copy(obufs[b], out_hbm.at[pl.ds(o, 8192)], ssem)

    def wait_store(b):
        pltpu.make_async_copy(
            obufs[b], out_hbm.at[pl.ds(0, 8192)], ssem).wait()

    def transpose(b, nk):
        def body(k, carry):
            for half in range(2):
                vcol = jnp.full((16,), 0, jnp.int32) + (2 * k + half)
                for g in range(4):
                    vec = plsc.load_gather(gbufs[b], [rows4[g], vcol])
                    obufs[b][pl.ds(k * 128 + half * 64 + g * 16, 16)] = vec
            return carry

        lax.fori_loop(0, nk, body, 0)

    fire_gather(0, 0)

    def outer(k2, carry):
        t = k2 * 2
        for p in range(2):
            tt = t + p

            @pl.when(tt + 1 < cnt)
            def _():
                fire_gather(tt + 1, 1 - p)

            @pl.when(tt < cnt)
            def _():
                wait_gather(p)

            @pl.when(jnp.logical_and(tt >= 2, tt < cnt + 2))
            def _():
                wait_store(p)

            @pl.when(tt < cnt)
            def _():
                transpose(p, 64)
                fire_store(tt, p)

        return carry

    lax.fori_loop(0, A_ITERS, outer, 0)

    @pl.when(wid == 31)
    def _():
        pltpu.sync_copy(tail_hbm, gbufs[0].at[:, pl.ds(0, 128)])
        transpose(0, 32)
        pltpu.sync_copy(
            obufs[0].at[pl.ds(0, 4096)],
            out_hbm.at[pl.ds(NC_FULL * 8192, 4096)])


def _gather_body(idx_hbm, table_hbm, out_hbm, idx_v, gbufs, obufs, gsem, ssem):
    wid = lax.axis_index("s") * 2 + lax.axis_index("c")
    w4 = wid * BLK_PER_W
    pltpu.sync_copy(idx_hbm.at[:, pl.ds(w4, BLK_PER_W), :], idx_v)

    iota = lax.iota(jnp.int32, 16)
    e8s = [(iota + g * 16) // 8 for g in range(4)]
    ers = [lax.rem(iota + g * 16, 8) for g in range(4)]

    def fire_gather(k, b):
        f = k // BLK_PER_W
        j = lax.rem(k, BLK_PER_W)
        pltpu.async_copy(table_hbm.at[idx_v.at[f, j]], gbufs[b], gsem)

    def wait_gather(b):
        pltpu.make_async_copy(
            table_hbm.at[idx_v.at[0, 0]], gbufs[b], gsem).wait()

    def fire_store(k, b):
        f = k // BLK_PER_W
        blk = w4 + lax.rem(k, BLK_PER_W)
        pltpu.async_copy(
            obufs[b].at[:, :, pl.ds(0, 128)], out_hbm.at[f, :, blk], ssem)

    def wait_store(b):
        pltpu.make_async_copy(
            obufs[b].at[:, :, pl.ds(0, 128)], out_hbm.at[0, :, 0], ssem).wait()

    def select(b):
        def inner(c, carry):
            cs = jnp.full((16,), 0, jnp.int32) + c
            for g in range(4):
                vec = gbufs[b][c, pl.ds(g * 16, 16)]
                plsc.store_scatter(obufs[b], [e8s[g], ers[g], cs], vec)
            return carry

        lax.fori_loop(0, 128, inner, 0)

    fire_gather(0, 0)

    def outer(k2, carry):
        k = k2 * 2
        for p in range(2):
            kk = k + p

            @pl.when(kk + 1 < ITEMS)
            def _():
                fire_gather(kk + 1, 1 - p)

            wait_gather(p)

            @pl.when(kk >= 2)
            def _():
                wait_store(p)

            select(p)
            fire_store(kk, p)
        return carry

    lax.fori_loop(0, ITEMS // 2, outer, 0)
    wait_store(0)
    wait_store(1)


def kernel(indices, E):
    mesh = plsc.VectorSubcoreMesh(core_axis_name="c", subcore_axis_name="s")

    relayout = pl.kernel(
        _relayout_body,
        out_type=jax.ShapeDtypeStruct((VOCAB * EMBED,), jnp.float32),
        mesh=mesh,
        scratch_types=[
            [pltpu.VMEM((EMBED, GPAD), jnp.float32) for _ in range(2)],
            [pltpu.VMEM((8192,), jnp.float32) for _ in range(2)],
            pltpu.SemaphoreType.DMA,
            pltpu.SemaphoreType.DMA,
        ],
        compiler_params=pltpu.CompilerParams(
            use_tc_tiling_on_sc=True, needs_layout_passes=False),
    )
    tail = jnp.pad(jnp.transpose(E[VOCAB - 64:]), ((0, 0), (0, 64)))
    table_lin = relayout(jnp.transpose(E), tail).reshape(VOCAB, EMBED)

    idx3 = jnp.transpose(indices).reshape(FIELDS, NBLK, 128).astype(jnp.int32)
    run = pl.kernel(
        _gather_body,
        out_type=jax.ShapeDtypeStruct((FIELDS, 8, NBLK, 8, 128), jnp.float32),
        mesh=mesh,
        scratch_types=[
            pltpu.VMEM((FIELDS, BLK_PER_W, 128), jnp.int32),
            [pltpu.VMEM((128, EMBED), jnp.float32) for _ in range(2)],
            [pltpu.VMEM((8, 8, GPAD), jnp.float32) for _ in range(2)],
            pltpu.SemaphoreType.DMA,
            pltpu.SemaphoreType.DMA,
        ],
        compiler_params=pltpu.CompilerParams(
            use_tc_tiling_on_sc=False, needs_layout_passes=False),
    )
    p_out = run(idx3, table_lin)
    return p_out.transpose(2, 4, 0, 1, 3).reshape(BATCH, FIELDS, EMBED)


# speedup vs baseline: 2.3018x; 2.3018x over previous
import jax
import jax.numpy as jnp
from jax import lax
from jax.experimental import pallas as pl
from jax.experimental.pallas import tpu as pltpu
from jax.experimental.pallas import tpu_sc as plsc

VOCAB = 1000000
BATCH = 16384
FIELDS = 26
EMBED = 64
NUM_WORKERS = 32
NBLK = BATCH // 128
BLK_PER_W = NBLK // NUM_WORKERS
ITEMS = FIELDS * BLK_PER_W
GPAD = 129


def _body(idx_hbm, table_hbm, out_hbm, idx_v, gbufs, obufs, gsem, ssem):
    wid = lax.axis_index("s") * 2 + lax.axis_index("c")
    w4 = wid * BLK_PER_W
    pltpu.sync_copy(idx_hbm.at[:, pl.ds(w4, BLK_PER_W), :], idx_v)

    iota = lax.iota(jnp.int32, 16)
    e8s = [(iota + g * 16) // 8 for g in range(4)]
    ers = [lax.rem(iota + g * 16, 8) for g in range(4)]

    def fire_gather(k, b):
        f = k // BLK_PER_W
        j = lax.rem(k, BLK_PER_W)
        pltpu.async_copy(table_hbm.at[idx_v.at[f, j]], gbufs[b], gsem)

    def wait_gather(b):
        pltpu.make_async_copy(
            table_hbm.at[idx_v.at[0, 0]], gbufs[b], gsem).wait()

    def fire_store(k, b):
        f = k // BLK_PER_W
        blk = w4 + lax.rem(k, BLK_PER_W)
        pltpu.async_copy(
            obufs[b].at[:, :, pl.ds(0, 128)], out_hbm.at[f, :, blk], ssem)

    def wait_store(b):
        pltpu.make_async_copy(
            obufs[b].at[:, :, pl.ds(0, 128)], out_hbm.at[0, :, 0], ssem).wait()

    def select(b):
        def inner(c4, carry):
            c0 = c4 * 4
            vecs = []
            for q in range(4):
                for g in range(4):
                    vecs.append(gbufs[b][c0 + q, pl.ds(g * 16, 16)])
            for q in range(4):
                cs = jnp.full((16,), 0, jnp.int32) + (c0 + q)
                for g in range(4):
                    plsc.store_scatter(
                        obufs[b], [e8s[g], ers[g], cs], vecs[q * 4 + g])
            return carry

        lax.fori_loop(0, 32, inner, 0)

    fire_gather(0, 0)

    def outer(k2, carry):
        k = k2 * 2
        for p in range(2):
            kk = k + p

            @pl.when(kk + 1 < ITEMS)
            def _():
                fire_gather(kk + 1, 1 - p)

            wait_gather(p)

            @pl.when(kk >= 2)
            def _():
                wait_store(p)

            select(p)
            fire_store(kk, p)
        return carry

    lax.fori_loop(0, ITEMS // 2, outer, 0)
    wait_store(0)
    wait_store(1)


def kernel(indices, E):
    idx3 = jnp.transpose(indices).reshape(FIELDS, NBLK, 128).astype(jnp.int32)
    mesh = plsc.VectorSubcoreMesh(core_axis_name="c", subcore_axis_name="s")
    run = pl.kernel(
        _body,
        out_type=jax.ShapeDtypeStruct((FIELDS, 8, NBLK, 8, 128), jnp.float32),
        mesh=mesh,
        scratch_types=[
            pltpu.VMEM((FIELDS, BLK_PER_W, 128), jnp.int32),
            [pltpu.VMEM((128, EMBED), jnp.float32) for _ in range(2)],
            [pltpu.VMEM((8, 8, GPAD), jnp.float32) for _ in range(2)],
            pltpu.SemaphoreType.DMA,
            pltpu.SemaphoreType.DMA,
        ],
        compiler_params=pltpu.CompilerParams(
            use_tc_tiling_on_sc=False, needs_layout_passes=False),
    )
    p_out = run(idx3, E)
    return p_out.transpose(2, 4, 0, 1, 3).reshape(BATCH, FIELDS, EMBED)
